# baseline (device time: 19784 ns/iter reference)
import jax
import jax.numpy as jnp
from jax import lax
from jax.experimental import pallas as pl
from jax.experimental.pallas import tpu as pltpu

N_DEV = 16
N_STEPS = 4
N_IDX = 512
ROWS_PER = 2048
D = 256
N_RAILS = 4
Q = N_IDX // N_RAILS

BASE_MASKS = (1, 3, 4, 8)
MASKS = tuple(
    tuple(BASE_MASKS[(s + r) % N_STEPS] for s in range(N_STEPS))
    for r in range(N_RAILS)
)


def kernel(table, idx):
    idx2 = idx.reshape(N_IDX, 1)

    def body(table_ref, idx_ref, out_ref, acc_ref, recv_ref, send_sems,
             recv_sems):
        my = lax.axis_index("i")

        barrier_sem = pltpu.get_barrier_semaphore()
        for m in BASE_MASKS:
            pl.semaphore_signal(
                barrier_sem,
                inc=1,
                device_id=(my ^ m,),
                device_id_type=pl.DeviceIdType.MESH,
            )

        table_bf16 = table_ref[:, :].astype(jnp.bfloat16)

        def partial_quarter(r):
            local = idx_ref[pl.ds(r * Q, Q), :] - my * ROWS_PER
            cols = lax.broadcasted_iota(jnp.int16, (Q, ROWS_PER), 1)
            onehot = (cols == local.astype(jnp.int16)).astype(jnp.bfloat16)
            acc = lax.dot_general(
                onehot,
                table_bf16,
                (((1,), (0,)), ((), ())),
                preferred_element_type=jnp.float32,
            )
            acc_ref[r, 0] = acc.astype(jnp.bfloat16)

        def start(rail, s, src_pp):
            rdma = pltpu.make_async_remote_copy(
                src_ref=acc_ref.at[rail, src_pp],
                dst_ref=recv_ref.at[s, rail],
                send_sem=send_sems.at[s, rail],
                recv_sem=recv_sems.at[s, rail],
                device_id=(my ^ MASKS[rail][s],),
                device_id_type=pl.DeviceIdType.MESH,
            )
            rdma.start()
            return rdma

        sends = {r: [None] * N_STEPS for r in range(N_RAILS)}
        partial_quarter(3)
        pl.semaphore_wait(barrier_sem, N_STEPS)
        sends[3][0] = start(3, 0, 0)
        for r in (0, 1, 2):
            partial_quarter(r)
            sends[r][0] = start(r, 0, 0)

        for s in range(N_STEPS):
            pp, nxt = s % 2, (s + 1) % 2
            slow = (3 - s) % N_RAILS
            for rail in [r for r in range(N_RAILS) if r != slow] + [slow]:
                sends[rail][s].wait_recv()
                if s >= 1:
                    sends[rail][s - 1].wait_send()
                summed = acc_ref[rail, pp] + recv_ref[s, rail]
                if s + 1 < N_STEPS:
                    acc_ref[rail, nxt] = summed
                    sends[rail][s + 1] = start(rail, s + 1, nxt)
                else:
                    out_ref[pl.ds(rail * Q, Q), :] = summed

        for rail in range(N_RAILS):
            sends[rail][N_STEPS - 1].wait_send()

    return pl.pallas_call(
        body,
        out_shape=jax.ShapeDtypeStruct((N_IDX, D), jnp.bfloat16),
        in_specs=[
            pl.BlockSpec(memory_space=pltpu.VMEM),
            pl.BlockSpec(memory_space=pltpu.VMEM),
        ],
        out_specs=pl.BlockSpec(memory_space=pltpu.VMEM),
        scratch_shapes=[
            pltpu.VMEM((N_RAILS, 2, Q, D), jnp.bfloat16),
            pltpu.VMEM((N_STEPS, N_RAILS, Q, D), jnp.bfloat16),
            pltpu.SemaphoreType.DMA((N_STEPS, N_RAILS)),
            pltpu.SemaphoreType.DMA((N_STEPS, N_RAILS)),
        ],
        compiler_params=pltpu.CompilerParams(collective_id=0),
    )(table, idx2)


# device time: 19403 ns/iter; 1.0196x vs baseline; 1.0196x over previous
import jax
import jax.numpy as jnp
from jax import lax
from jax.experimental import pallas as pl
from jax.experimental.pallas import tpu as pltpu

N_DEV = 16
N_STEPS = 4
N_IDX = 512
ROWS_PER = 2048
D = 256
N_RAILS = 4
Q = N_IDX // N_RAILS

BASE_MASKS = (1, 3, 4, 8)
MASKS = tuple(
    tuple(BASE_MASKS[(s + r) % N_STEPS] for s in range(N_STEPS))
    for r in range(N_RAILS)
)


def kernel(table, idx):
    idx2 = idx.reshape(N_IDX, 1)

    def body(table_ref, idx_ref, out_ref, acc_ref, recv_ref, send_sems,
             recv_sems):
        my = lax.axis_index("i")

        barrier_sem = pltpu.get_barrier_semaphore()
        for m in BASE_MASKS:
            pl.semaphore_signal(
                barrier_sem,
                inc=1,
                device_id=(my ^ m,),
                device_id_type=pl.DeviceIdType.MESH,
            )

        table_bf16 = table_ref[:, :].astype(jnp.bfloat16)

        def partial_quarter(r):
            local = idx_ref[pl.ds(r * Q, Q), :] - my * ROWS_PER
            cols = lax.broadcasted_iota(jnp.int16, (Q, ROWS_PER), 1)
            onehot = (cols == local.astype(jnp.int16)).astype(jnp.bfloat16)
            acc = lax.dot_general(
                onehot,
                table_bf16,
                (((1,), (0,)), ((), ())),
                preferred_element_type=jnp.float32,
            )
            acc_ref[r, 0] = acc.astype(jnp.bfloat16)

        def start(rail, s, src_pp):
            rdma = pltpu.make_async_remote_copy(
                src_ref=acc_ref.at[rail, src_pp],
                dst_ref=recv_ref.at[s, rail],
                send_sem=send_sems.at[s, rail],
                recv_sem=recv_sems.at[s, rail],
                device_id=(my ^ MASKS[rail][s],),
                device_id_type=pl.DeviceIdType.MESH,
            )
            rdma.start()
            return rdma

        sends = {r: [None] * N_STEPS for r in range(N_RAILS)}
        partial_quarter(0)
        pl.semaphore_wait(barrier_sem, N_STEPS)
        sends[0][0] = start(0, 0, 0)
        for r in range(1, N_RAILS):
            partial_quarter(r)
            sends[r][0] = start(r, 0, 0)

        for s in range(N_STEPS):
            pp, nxt = s % 2, (s + 1) % 2
            slow = (3 - s) % N_RAILS
            for rail in [r for r in range(N_RAILS) if r != slow] + [slow]:
                sends[rail][s].wait_recv()
                if s >= 1:
                    sends[rail][s - 1].wait_send()
                summed = acc_ref[rail, pp] + recv_ref[s, rail]
                if s + 1 < N_STEPS:
                    acc_ref[rail, nxt] = summed
                    sends[rail][s + 1] = start(rail, s + 1, nxt)
                else:
                    out_ref[pl.ds(rail * Q, Q), :] = summed

        for rail in range(N_RAILS):
            sends[rail][N_STEPS - 1].wait_send()

    return pl.pallas_call(
        body,
        out_shape=jax.ShapeDtypeStruct((N_IDX, D), jnp.bfloat16),
        in_specs=[
            pl.BlockSpec(memory_space=pltpu.VMEM),
            pl.BlockSpec(memory_space=pltpu.VMEM),
        ],
        out_specs=pl.BlockSpec(memory_space=pltpu.VMEM),
        scratch_shapes=[
            pltpu.VMEM((N_RAILS, 2, Q, D), jnp.bfloat16),
            pltpu.VMEM((N_STEPS, N_RAILS, Q, D), jnp.bfloat16),
            pltpu.SemaphoreType.DMA((N_STEPS, N_RAILS)),
            pltpu.SemaphoreType.DMA((N_STEPS, N_RAILS)),
        ],
        compiler_params=pltpu.CompilerParams(collective_id=0),
    )(table, idx2)
